# Initial kernel scaffold; baseline (speedup 1.0000x reference)
import jax
import jax.numpy as jnp
from jax.experimental import pallas as pl


def kernel(x, edge_index, edge_attr, W1, att_src1, att_dst1, lin_e1, att_e1, bias1, g1, be1, W2, att_src2, att_dst2, lin_e2, att_e2, bias2, g2, be2, Wf, bf, gf, bff):
    raise NotImplementedError("write your pallas kernel here")


# speedup vs baseline: 1.2626x; 1.2626x over previous
import functools

import jax
import jax.numpy as jnp
from jax.experimental import pallas as pl
from jax.experimental.pallas import tpu as pltpu

N = 50000
E = 800000
IN_CH = 20
HID = 16
HEADS = 4
LATENT = 128
EDGE_DIM = 3


def _leaky(x):
    return jnp.maximum(x, 0.2 * x)


def _ln(x, g, b):
    m = jnp.mean(x, axis=-1, keepdims=True)
    v = jnp.var(x, axis=-1, keepdims=True)
    return (x - m) / jnp.sqrt(v + 1e-5) * g + b


def _gat_jnp(x, src, dst, ea, ea_mean, W, a_s, a_d, We, a_e, bias, H, C):
    n = x.shape[0]
    xt = (x @ W).reshape(n, H, C)
    al_s = jnp.sum(xt * a_s, axis=-1)
    al_d = jnp.sum(xt * a_d, axis=-1)
    al_e = jnp.sum((ea @ We).reshape(-1, H, C) * a_e, axis=-1)
    al_e_self = jnp.sum((ea_mean @ We).reshape(1, H, C) * a_e, axis=-1)
    w = jnp.exp(_leaky(al_s[src] + al_d[dst] + al_e))
    num = jax.ops.segment_sum(xt[src] * w[:, :, None], dst, num_segments=n)
    den = jax.ops.segment_sum(w, dst, num_segments=n)
    w_self = jnp.exp(_leaky(al_s + al_d + al_e_self))
    num = num + xt * w_self[:, :, None]
    den = den + w_self
    out = num / den[:, :, None]
    return out.reshape(n, H * C) if H > 1 else out[:, 0], xt


def _final_kernel(h_ref, wf_ref, bf_ref, gf_ref, bff_ref, o_ref, acc_ref):
    i = pl.program_id(0)
    nb = pl.num_programs(0)
    blk = h_ref[...]
    bsum = jnp.sum(blk, axis=0, keepdims=True)
    bmax = jnp.max(blk, axis=0, keepdims=True)

    @pl.when(i == 0)
    def _():
        acc_ref[0:1, 0:HID] = bsum
        acc_ref[1:2, 0:HID] = bmax

    @pl.when(i > 0)
    def _():
        acc_ref[0:1, 0:HID] = acc_ref[0:1, 0:HID] + bsum
        acc_ref[1:2, 0:HID] = jnp.maximum(acc_ref[1:2, 0:HID], bmax)

    @pl.when(i == nb - 1)
    def _():
        xm = acc_ref[0:1, 0:HID] / N
        xx = acc_ref[1:2, 0:HID]
        xg = jnp.concatenate([xm, xx], axis=1)
        out = xg @ wf_ref[...] + bf_ref[...]
        out = jnp.maximum(out, 0.0)
        o_ref[...] = _ln(out, gf_ref[...], bff_ref[...])


def _final_stage(h, Wf, bf, gf, bff):
    BLK = 2000
    return pl.pallas_call(
        _final_kernel,
        grid=(N // BLK,),
        in_specs=[
            pl.BlockSpec((BLK, HID), lambda i: (i, 0)),
            pl.BlockSpec((2 * HID, LATENT), lambda i: (0, 0)),
            pl.BlockSpec((1, LATENT), lambda i: (0, 0)),
            pl.BlockSpec((1, LATENT), lambda i: (0, 0)),
            pl.BlockSpec((1, LATENT), lambda i: (0, 0)),
        ],
        out_specs=pl.BlockSpec((1, LATENT), lambda i: (0, 0)),
        out_shape=jax.ShapeDtypeStruct((1, LATENT), jnp.float32),
        scratch_shapes=[pltpu.VMEM((2, 128), jnp.float32)],
    )(h, Wf, bf.reshape(1, -1), gf.reshape(1, -1), bff.reshape(1, -1))


def kernel(x, edge_index, edge_attr, W1, att_src1, att_dst1, lin_e1, att_e1,
           bias1, g1, be1, W2, att_src2, att_dst2, lin_e2, att_e2, bias2, g2,
           be2, Wf, bf, gf, bff):
    src, dst = edge_index[0], edge_index[1]
    ea_mean = jnp.mean(edge_attr, axis=0, keepdims=True)
    h, _ = _gat_jnp(x, src, dst, edge_attr, ea_mean, W1, att_src1, att_dst1,
                    lin_e1, att_e1, bias1, HEADS, HID)
    h = h + bias1
    h = jax.nn.elu(h)
    h = _ln(h, g1, be1)
    h, _ = _gat_jnp(h, src, dst, edge_attr, ea_mean, W2, att_src2, att_dst2,
                    lin_e2, att_e2, bias2, 1, HID)
    h = h + bias2
    h = jax.nn.elu(h)
    h = _ln(h, g2, be2)
    return _final_stage(h, Wf, bf, gf, bff)


# speedup vs baseline: 16.3007x; 12.9108x over previous
import functools

import jax
import jax.numpy as jnp
from jax import lax
from jax.experimental import pallas as pl
from jax.experimental.pallas import tpu as pltpu
from jax.experimental.pallas import tpu_sc as plsc

N = 50000
E = 800000
IN_CH = 20
HID = 16
HEADS = 4
LATENT = 128
EDGE_DIM = 3

NC = 2
NS = 16
L = 16

ROWH = 24
ROW2 = 32
ACC = 24

NP = 50176
RPT = NP // NS
CB = 112
KE1 = 80
KE2 = 80
NCHUNK = 625
EH = E // NC

_i32 = jnp.int32
_f32 = jnp.float32


def _iota16():
    return lax.iota(_i32, L)


def _bcast16(x):
    return jnp.full((L,), x, _i32)


def _prep_nodes_kernel(x_ref, w1_ref, a_ref, tabH_ref, tabD_ref):
    xt = jnp.dot(x_ref[...], w1_ref[...],
                 preferred_element_type=_f32)
    al = jnp.dot(xt, a_ref[...], preferred_element_type=_f32)
    b = xt.shape[0]
    z6 = jnp.zeros((b, 6), _f32)
    z12 = jnp.zeros((b, 12), _f32)
    ts = [jnp.concatenate([xt[:, h * HID:(h + 1) * HID], al[:, h:h + 1],
                           al[:, 4 + h:5 + h], z6], axis=1)
          for h in range(HEADS)]
    tabH_ref[...] = jnp.stack(ts, axis=0)
    tabD_ref[...] = jnp.concatenate([al[:, 4:8], z12], axis=1)


def _prep_nodes(x, W1, A):
    B = 2000
    return pl.pallas_call(
        _prep_nodes_kernel,
        grid=(N // B,),
        in_specs=[
            pl.BlockSpec((B, IN_CH), lambda i: (i, 0)),
            pl.BlockSpec((IN_CH, 64), lambda i: (0, 0)),
            pl.BlockSpec((64, 8), lambda i: (0, 0)),
        ],
        out_specs=[
            pl.BlockSpec((4, B, ROWH), lambda i: (0, i, 0)),
            pl.BlockSpec((B, 16), lambda i: (i, 0)),
        ],
        out_shape=[
            jax.ShapeDtypeStruct((4, NP, ROWH), _f32),
            jax.ShapeDtypeStruct((NP, 16), _f32),
        ],
    )(x, W1, A)


def _prep_edges_kernel(ea_ref, src_ref, le1_ref, ae1_ref, le2_ref, ae2_ref,
                       ale1_ref, ale2_ref, selfc_ref, srcx_ref, acc_ref):
    i = pl.program_id(0)
    nb = pl.num_programs(0)
    ea = ea_ref[...]
    b1 = jnp.dot(le1_ref[...], ae1_ref[...], preferred_element_type=_f32)
    b2 = jnp.dot(le2_ref[...], ae2_ref[...], preferred_element_type=_f32)
    ale_t = lax.dot_general(b1, ea, (((0,), (1,)), ((), ())),
                            preferred_element_type=_f32)
    ale2 = jnp.dot(ea, b2, preferred_element_type=_f32)
    ale1_ref[...] = ale_t
    ale2_ref[...] = ale2
    srcv = src_ref[...][:, 0]
    srcx_ref[...] = jnp.stack([srcv + h * NP for h in range(HEADS)], axis=0)

    @pl.when(i == 0)
    def _():
        acc_ref[...] = jnp.zeros_like(acc_ref)

    acc_ref[0:1, 0:EDGE_DIM] = acc_ref[0:1, 0:EDGE_DIM] + jnp.sum(
        ea, axis=0, keepdims=True)

    @pl.when(i == nb - 1)
    def _():
        mean = acc_ref[0:1, 0:EDGE_DIM] / E
        sl1 = jnp.dot(mean, b1, preferred_element_type=_f32)
        sl2 = jnp.dot(mean, b2, preferred_element_type=_f32)
        z11 = jnp.zeros((1, 11), _f32)
        selfc_ref[...] = jnp.concatenate([sl1, sl2, z11], axis=1)


def _prep_edges(ea, src, lin_e1, Ae1, lin_e2, ae2):
    B = 6400
    return pl.pallas_call(
        _prep_edges_kernel,
        grid=(E // B,),
        in_specs=[
            pl.BlockSpec((B, EDGE_DIM), lambda i: (i, 0)),
            pl.BlockSpec((B, 1), lambda i: (i, 0)),
            pl.BlockSpec((EDGE_DIM, 64), lambda i: (0, 0)),
            pl.BlockSpec((64, 4), lambda i: (0, 0)),
            pl.BlockSpec((EDGE_DIM, 16), lambda i: (0, 0)),
            pl.BlockSpec((16, 1), lambda i: (0, 0)),
        ],
        out_specs=[
            pl.BlockSpec((4, B), lambda i: (0, i)),
            pl.BlockSpec((B, 1), lambda i: (i, 0)),
            pl.BlockSpec((1, 16), lambda i: (0, 0)),
            pl.BlockSpec((4, B), lambda i: (0, i)),
        ],
        out_shape=[
            jax.ShapeDtypeStruct((4, E), _f32),
            jax.ShapeDtypeStruct((E, 1), _f32),
            jax.ShapeDtypeStruct((1, 16), _f32),
            jax.ShapeDtypeStruct((4, E), _i32),
        ],
        scratch_shapes=[pltpu.VMEM((1, 128), _f32)],
    )(ea, src.reshape(E, 1), lin_e1, Ae1, lin_e2, ae2)


def _sc_mesh():
    return plsc.VectorSubcoreMesh(core_axis_name="c", subcore_axis_name="s",
                                  num_cores=NC, num_subcores=NS)


_SC_PARAMS = pltpu.CompilerParams(
    needs_layout_passes=False, use_tc_tiling_on_sc=False)


def _zero_pad_cols(upd, nrows):
    z16v = jnp.zeros((L,), _f32)
    for g in range(nrows // L):
        rows = _iota16() + g * L
        for col in range(HID + 1, ACC):
            plsc.store_scatter(upd, [rows, _bcast16(col)], z16v)


def _run_edge1(tabH2d, tabD, srcx, dst, ale1, zeros24):
    @functools.partial(
        pl.kernel,
        out_type=jax.ShapeDtypeStruct((HEADS, NP, ACC), _f32),
        mesh=_sc_mesh(),
        compiler_params=_SC_PARAMS,
        scratch_types=[
            pltpu.VMEM((KE1,), _i32),
            pltpu.VMEM((KE1,), _i32),
            pltpu.VMEM((KE1,), _i32),
            pltpu.VMEM((KE1,), _f32),
            pltpu.VMEM((KE1, ROWH), _f32),
            pltpu.VMEM((KE1, 16), _f32),
            pltpu.VMEM((KE1, ACC), _f32),
            pltpu.SemaphoreType.DMA,
            pltpu.VMEM((CB, ACC), _f32),
            pltpu.VMEM_SHARED((NP, ACC), _f32),
        ],
    )
    def k(tabH_h, tabD_h, srcx_h, dst_h, ale1_h, z24_h, out_h,
          srcb, dstb, srcadj, aleb, rowsS, rowsD, upd, sem, zbuf, acc):
        c = lax.axis_index("c")
        s = lax.axis_index("s")

        _zero_pad_cols(upd, KE1)
        pltpu.sync_copy(z24_h.at[pl.ds(s * RPT, CB), :], zbuf)
        base0 = s * (E // NS)

        for p in range(2):
            head = 2 * c + p
            for q in range(RPT // CB):
                pltpu.sync_copy(zbuf, acc.at[pl.ds(s * RPT + q * CB, CB), :])
            plsc.subcore_barrier()

            def _echunk(j, carry):
                base = base0 + j * KE1
                pltpu.sync_copy(srcx_h.at[pl.ds(head * E + base, KE1)],
                                srcadj)
                pltpu.sync_copy(dst_h.at[pl.ds(base, KE1)], dstb)
                pltpu.sync_copy(ale1_h.at[pl.ds(head * E + base, KE1)], aleb)
                pltpu.sync_copy(tabH_h.at[srcadj], rowsS)
                pltpu.sync_copy(tabD_h.at[dstb], rowsD)
                for g in range(KE1 // L):
                    rows = _iota16() + g * L
                    als = plsc.load_gather(rowsS, [rows, _bcast16(HID)])
                    ald = plsc.load_gather(rowsD, [rows, _bcast16(p) + 2 * c])
                    alev = aleb[pl.ds(g * L, L)]
                    lg = als + ald + alev
                    w = jnp.exp(jnp.maximum(lg, 0.2 * lg))
                    plsc.store_scatter(upd, [rows, _bcast16(HID)], w)
                    for f in range(HID):
                        xv = plsc.load_gather(rowsS, [rows, _bcast16(f)])
                        plsc.store_scatter(upd, [rows, _bcast16(f)], xv * w)
                pltpu.sync_copy(upd, acc.at[dstb], add=True)
                return carry

            lax.fori_loop(0, NCHUNK, _echunk, 0)
            plsc.subcore_barrier()
            for q in range(RPT // CB):
                r0 = s * RPT + q * CB
                pltpu.sync_copy(acc.at[pl.ds(r0, CB), :], zbuf)
                pltpu.sync_copy(zbuf, out_h.at[head, pl.ds(r0, CB), :])
            plsc.subcore_barrier()
            pltpu.sync_copy(z24_h.at[pl.ds(s * RPT, CB), :], zbuf)

    return k(tabH2d, tabD, srcx, dst, ale1, zeros24)


def _ln(x, g, b):
    m = jnp.mean(x, axis=-1, keepdims=True)
    v = jnp.var(x, axis=-1, keepdims=True)
    return (x - m) / jnp.sqrt(v + 1e-5) * g + b


def _elu(x):
    return jnp.where(x > 0, x, jnp.exp(x) - 1.0)


def _mid_kernel(acc_ref, tabH_ref, selfc_ref, rep_ref, b1_ref, g1_ref,
                be1_ref, w2_ref, a2_ref, tab2_ref, tabD2_ref):
    num = jnp.concatenate([acc_ref[h, :, 0:HID] for h in range(HEADS)],
                          axis=1)
    den4 = jnp.concatenate([acc_ref[h, :, HID:HID + 1] for h in range(HEADS)],
                           axis=1)
    xt = jnp.concatenate([tabH_ref[h, :, 0:HID] for h in range(HEADS)],
                         axis=1)
    als4 = jnp.concatenate([tabH_ref[h, :, HID:HID + 1] for h in range(HEADS)],
                           axis=1)
    ald4 = jnp.concatenate(
        [tabH_ref[h, :, HID + 1:HID + 2] for h in range(HEADS)], axis=1)
    lg = als4 + ald4 + selfc_ref[0:1, 0:4]
    w4 = jnp.exp(jnp.maximum(lg, 0.2 * lg))
    w64 = jnp.dot(w4, rep_ref[...], preferred_element_type=_f32)
    num = num + xt * w64
    den = jnp.dot(den4 + w4, rep_ref[...], preferred_element_type=_f32)
    h = num / den + b1_ref[...]
    h = _elu(h)
    h = _ln(h, g1_ref[...], be1_ref[...])
    xt2 = jnp.dot(h, w2_ref[...], preferred_element_type=_f32)
    al2 = jnp.dot(xt2, a2_ref[...], preferred_element_type=_f32)
    b = xt2.shape[0]
    tab2_ref[...] = jnp.concatenate(
        [xt2, al2[:, 0:1], al2[:, 1:2], jnp.zeros((b, 14), _f32)], axis=1)
    tabD2_ref[...] = jnp.concatenate(
        [al2[:, 1:2], jnp.zeros((b, 15), _f32)], axis=1)


def _mid(acc1, tabH, selfc, Rep, bias1, g1, be1, W2, A2):
    B = 2000
    return pl.pallas_call(
        _mid_kernel,
        grid=(N // B,),
        in_specs=[
            pl.BlockSpec((4, B, ACC), lambda i: (0, i, 0)),
            pl.BlockSpec((4, B, ROWH), lambda i: (0, i, 0)),
            pl.BlockSpec((1, 16), lambda i: (0, 0)),
            pl.BlockSpec((4, 64), lambda i: (0, 0)),
            pl.BlockSpec((1, 64), lambda i: (0, 0)),
            pl.BlockSpec((1, 64), lambda i: (0, 0)),
            pl.BlockSpec((1, 64), lambda i: (0, 0)),
            pl.BlockSpec((64, HID), lambda i: (0, 0)),
            pl.BlockSpec((HID, 2), lambda i: (0, 0)),
        ],
        out_specs=[
            pl.BlockSpec((B, ROW2), lambda i: (i, 0)),
            pl.BlockSpec((B, 16), lambda i: (i, 0)),
        ],
        out_shape=[
            jax.ShapeDtypeStruct((NP, ROW2), _f32),
            jax.ShapeDtypeStruct((NP, 16), _f32),
        ],
    )(acc1, tabH, selfc, Rep, bias1.reshape(1, -1), g1.reshape(1, -1),
      be1.reshape(1, -1), W2, A2)


def _run_edge2(tab2, tabD2, src, dst, ale2, zeros24):
    @functools.partial(
        pl.kernel,
        out_type=jax.ShapeDtypeStruct((NC, NP, ACC), _f32),
        mesh=_sc_mesh(),
        compiler_params=_SC_PARAMS,
        scratch_types=[
            pltpu.VMEM((KE2,), _i32),
            pltpu.VMEM((KE2,), _i32),
            pltpu.VMEM((KE2,), _f32),
            pltpu.VMEM((KE2, ROW2), _f32),
            pltpu.VMEM((KE2, 16), _f32),
            pltpu.VMEM((KE2, ACC), _f32),
            pltpu.SemaphoreType.DMA,
            pltpu.VMEM((CB, ACC), _f32),
            pltpu.VMEM_SHARED((NP, ACC), _f32),
        ],
    )
    def k(tab2_h, tabD2_h, src_h, dst_h, ale2_h, z24_h, out_h,
          srcb, dstb, aleb, rows2, rowsD2, upd, sem, zbuf, acc):
        c = lax.axis_index("c")
        s = lax.axis_index("s")

        _zero_pad_cols(upd, KE2)
        pltpu.sync_copy(z24_h.at[pl.ds(s * RPT, CB), :], zbuf)
        for q in range(RPT // CB):
            pltpu.sync_copy(zbuf, acc.at[pl.ds(s * RPT + q * CB, CB), :])
        plsc.subcore_barrier()

        base0 = s * (E // NS)

        def _echunk(j, carry):
            base = base0 + j * KE2
            pltpu.sync_copy(src_h.at[pl.ds(base, KE2)], srcb)
            pltpu.sync_copy(dst_h.at[pl.ds(base, KE2)], dstb)
            pltpu.sync_copy(ale2_h.at[pl.ds(base, KE2)], aleb)
            pltpu.sync_copy(tab2_h.at[srcb], rows2)
            pltpu.sync_copy(tabD2_h.at[dstb], rowsD2)
            for g in range(KE2 // L):
                rows = _iota16() + g * L
                als = plsc.load_gather(rows2, [rows, _bcast16(HID)])
                ald = plsc.load_gather(rowsD2, [rows, _bcast16(0)])
                alev = aleb[pl.ds(g * L, L)]
                lg = als + ald + alev
                w = jnp.exp(jnp.maximum(lg, 0.2 * lg))
                plsc.store_scatter(upd, [rows, _bcast16(HID)], w)
                for f in range(HID):
                    xv = plsc.load_gather(rows2, [rows, _bcast16(f)])
                    plsc.store_scatter(upd, [rows, _bcast16(f)], xv * w)
            pltpu.sync_copy(upd, acc.at[dstb], add=True)
            return carry

        lax.fori_loop(0, NCHUNK, _echunk, 0)
        plsc.subcore_barrier()
        for q in range(RPT // CB):
            r0 = s * RPT + q * CB
            pltpu.sync_copy(acc.at[pl.ds(r0, CB), :], zbuf)
            pltpu.sync_copy(zbuf, out_h.at[c, pl.ds(r0, CB), :])

    return k(tab2, tabD2, src, dst, ale2, zeros24)


def _final_kernel(acc_ref, tab2_ref, selfc_ref, b2_ref, g2_ref, be2_ref,
                  wf_ref, bf_ref, gf_ref, bff_ref, o_ref, pool_ref):
    i = pl.program_id(0)
    nb = pl.num_programs(0)
    xt2 = tab2_ref[:, 0:HID]
    lg = tab2_ref[:, HID:HID + 1] + tab2_ref[:, HID + 1:HID + 2] \
        + selfc_ref[0:1, 4:5]
    w = jnp.exp(jnp.maximum(lg, 0.2 * lg))
    num = 0.5 * (acc_ref[0, :, 0:HID] + acc_ref[1, :, 0:HID]) + xt2 * w
    den = 0.5 * (acc_ref[0, :, HID:HID + 1]
                 + acc_ref[1, :, HID:HID + 1]) + w
    h = num / den + b2_ref[...]
    h = _elu(h)
    h = _ln(h, g2_ref[...], be2_ref[...])
    bsum = jnp.sum(h, axis=0, keepdims=True)
    bmax = jnp.max(h, axis=0, keepdims=True)

    @pl.when(i == 0)
    def _():
        pool_ref[0:1, 0:HID] = bsum
        pool_ref[1:2, 0:HID] = bmax

    @pl.when(i > 0)
    def _():
        pool_ref[0:1, 0:HID] = pool_ref[0:1, 0:HID] + bsum
        pool_ref[1:2, 0:HID] = jnp.maximum(pool_ref[1:2, 0:HID], bmax)

    @pl.when(i == nb - 1)
    def _():
        xm = pool_ref[0:1, 0:HID] / N
        xx = pool_ref[1:2, 0:HID]
        xg = jnp.concatenate([xm, xx], axis=1)
        out = jnp.dot(xg, wf_ref[...], preferred_element_type=_f32)
        out = out + bf_ref[...]
        out = jnp.maximum(out, 0.0)
        o_ref[...] = _ln(out, gf_ref[...], bff_ref[...])


def _final(acc2, tab2, selfc, bias2, g2, be2, Wf, bf, gf, bff):
    B = 2000
    return pl.pallas_call(
        _final_kernel,
        grid=(N // B,),
        in_specs=[
            pl.BlockSpec((2, B, ACC), lambda i: (0, i, 0)),
            pl.BlockSpec((B, ROW2), lambda i: (i, 0)),
            pl.BlockSpec((1, 16), lambda i: (0, 0)),
            pl.BlockSpec((1, HID), lambda i: (0, 0)),
            pl.BlockSpec((1, HID), lambda i: (0, 0)),
            pl.BlockSpec((1, HID), lambda i: (0, 0)),
            pl.BlockSpec((2 * HID, LATENT), lambda i: (0, 0)),
            pl.BlockSpec((1, LATENT), lambda i: (0, 0)),
            pl.BlockSpec((1, LATENT), lambda i: (0, 0)),
            pl.BlockSpec((1, LATENT), lambda i: (0, 0)),
        ],
        out_specs=pl.BlockSpec((1, LATENT), lambda i: (0, 0)),
        out_shape=jax.ShapeDtypeStruct((1, LATENT), _f32),
        scratch_shapes=[pltpu.VMEM((2, 128), _f32)],
    )(acc2, tab2, selfc, bias2.reshape(1, -1), g2.reshape(1, -1),
      be2.reshape(1, -1), Wf, bf.reshape(1, -1), gf.reshape(1, -1),
      bff.reshape(1, -1))


def kernel(x, edge_index, edge_attr, W1, att_src1, att_dst1, lin_e1, att_e1,
           bias1, g1, be1, W2, att_src2, att_dst2, lin_e2, att_e2, bias2, g2,
           be2, Wf, bf, gf, bff):
    src, dst = edge_index[0], edge_index[1]

    A = jnp.zeros((64, 8), _f32)
    for h in range(HEADS):
        A = A.at[h * HID:(h + 1) * HID, h].set(att_src1[0, h, :])
        A = A.at[h * HID:(h + 1) * HID, 4 + h].set(att_dst1[0, h, :])
    Ae1 = jnp.zeros((64, 4), _f32)
    for h in range(HEADS):
        Ae1 = Ae1.at[h * HID:(h + 1) * HID, h].set(att_e1[0, h, :])
    ae2 = att_e2[0, 0, :].reshape(16, 1)
    A2 = jnp.concatenate([att_src2[0, 0, :].reshape(16, 1),
                          att_dst2[0, 0, :].reshape(16, 1)], axis=1)
    Rep = jnp.repeat(jnp.eye(4, dtype=_f32), HID, axis=1)

    zeros24 = jnp.zeros((NP, ACC), _f32)

    tabH, tabD = _prep_nodes(x, W1, A)
    ale1, ale2, selfc, srcx = _prep_edges(edge_attr, src, lin_e1, Ae1,
                                          lin_e2, ae2)

    acc1 = _run_edge1(tabH.reshape(4 * NP, ROWH), tabD,
                      srcx.reshape(4 * E), dst, ale1.reshape(4 * E), zeros24)
    tab2, tabD2 = _mid(acc1, tabH, selfc, Rep, bias1, g1, be1, W2, A2)
    acc2 = _run_edge2(tab2, tabD2, src, dst, ale2.reshape(E), zeros24)
    return _final(acc2, tab2, selfc, bias2, g2, be2, Wf, bf, gf, bff)
